# baseline (speedup 1.0000x reference)
import functools

import jax
import jax.numpy as jnp
from jax import lax
from jax.experimental import pallas as pl
from jax.experimental.pallas import tpu as pltpu
from jax.experimental.pallas import tpu_sc as plsc

_NC = 2
_NS = 16
_CHW = 50
_CHN = 400


@functools.lru_cache(maxsize=None)
def _make_agg(n_nodes: int, n_edges: int, width: int, gather: bool, ch: int):
    nw = _NC * _NS
    epw = n_edges // nw
    assert epw * nw == n_edges and epw % ch == 0
    nit = epw // ch
    cpt = nit
    assert nit >= 3
    npad = -(-n_nodes // (8 * _NS)) * (8 * _NS)
    rpt = npad // _NS

    mesh = plsc.VectorSubcoreMesh(
        core_axis_name="c", subcore_axis_name="s",
        num_cores=_NC, num_subcores=_NS)

    deep = gather and nit % 4 == 0 and nit >= 8

    def body(ei_hbm, tab_hbm, zeros_hbm, out_hbm,
             src_v, dst_v, rows_a, rows_b, rows_c, rows_d, acc_sh,
             gs0, gs1, gs2, gs3, ss0, ss1, ss2, ss3, sem_i):
        sem_a, sem_b = gs0, gs1
        c = lax.axis_index("c")
        s = lax.axis_index("s")
        wid = s * _NC + c
        idx = pltpu.async_copy(ei_hbm.at[1, wid], dst_v, sem_i)
        if gather:
            idx2 = pltpu.async_copy(ei_hbm.at[0, wid], src_v, sem_i)
        else:
            idx2 = None
            pltpu.sync_copy(tab_hbm, rows_a)
        pltpu.sync_copy(zeros_hbm.at[pl.ds(s * rpt, rpt)],
                        acc_sh.at[pl.ds(s * rpt, rpt)])
        idx.wait()
        if idx2 is not None:
            idx2.wait()
        plsc.subcore_barrier()

        if deep:
            bufs = [rows_a, rows_b, rows_c, rows_d]
            gsems = [gs0, gs1, gs2, gs3]
            ssems = [ss0, ss1, ss2, ss3]

            def gi(i, q):
                pltpu.async_copy(tab_hbm.at[src_v.at[i]], bufs[q], gsems[q])

            def gw(i, q):
                pltpu.make_async_copy(
                    tab_hbm.at[src_v.at[i]], bufs[q], gsems[q]).wait()

            def si(i, q):
                pltpu.async_copy(
                    bufs[q], acc_sh.at[dst_v.at[i]], ssems[q], add=True)

            def sw(i, q):
                pltpu.make_async_copy(
                    bufs[q], acc_sh.at[dst_v.at[i]], ssems[q]).wait()

            gi(0, 0)
            gi(1, 1)
            gw(0, 0); si(0, 0); gi(2, 2)
            gw(1, 1); si(1, 1); gi(3, 3)

            def quad(j, carry):
                i = 4 * j + 2
                for m in range(4):
                    q = (2 + m) % 4
                    gw(i + m, q)
                    si(i + m, q)
                    sw(i + m - 2, (q + 2) % 4)
                    gi(i + m + 2, (q + 2) % 4)
                return carry

            lax.fori_loop(0, (nit - 4) // 4, quad, 0)
            gw(nit - 2, 2); si(nit - 2, 2); sw(nit - 4, 0)
            gw(nit - 1, 3); si(nit - 1, 3); sw(nit - 3, 1)
            sw(nit - 2, 2)
            sw(nit - 1, 3)
        elif gather:
            def g_issue(i, buf, sem):
                pltpu.async_copy(tab_hbm.at[src_v.at[i]], buf, sem)

            def g_wait(i, buf, sem):
                pltpu.make_async_copy(tab_hbm.at[src_v.at[i]], buf, sem).wait()

            g_issue(0, rows_a, sem_a)

            def pair(j, carry):
                i = 2 * j
                g_issue(i + 1, rows_b, sem_b)
                g_wait(i, rows_a, sem_a)
                pltpu.sync_copy(rows_a, acc_sh.at[dst_v.at[i]], add=True)
                g_issue(i + 2, rows_a, sem_a)
                g_wait(i + 1, rows_b, sem_b)
                pltpu.sync_copy(rows_b, acc_sh.at[dst_v.at[i + 1]], add=True)
                return carry

            if nit % 2 == 1:
                lax.fori_loop(0, (nit - 1) // 2, pair, 0)
                g_wait(nit - 1, rows_a, sem_a)
                pltpu.sync_copy(rows_a, acc_sh.at[dst_v.at[nit - 1]], add=True)
            else:
                lax.fori_loop(0, nit // 2 - 1, pair, 0)
                g_issue(nit - 1, rows_b, sem_b)
                g_wait(nit - 2, rows_a, sem_a)
                pltpu.sync_copy(rows_a, acc_sh.at[dst_v.at[nit - 2]], add=True)
                g_wait(nit - 1, rows_b, sem_b)
                pltpu.sync_copy(rows_b, acc_sh.at[dst_v.at[nit - 1]], add=True)
        else:
            def step(i, carry):
                pltpu.sync_copy(rows_a, acc_sh.at[dst_v.at[i]], add=True)
                return carry

            lax.fori_loop(0, nit, step, 0)

        plsc.subcore_barrier()
        pltpu.sync_copy(acc_sh.at[pl.ds(s * rpt, rpt)],
                        out_hbm.at[c, pl.ds(s * rpt, rpt)])

    return pl.kernel(
        body,
        out_type=jax.ShapeDtypeStruct((_NC, npad, width), jnp.float32),
        mesh=mesh,
        scratch_types=[
            pltpu.VMEM((cpt, ch), jnp.int32),
            pltpu.VMEM((cpt, ch), jnp.int32),
            pltpu.VMEM((ch, width), jnp.float32),
            pltpu.VMEM((ch, width), jnp.float32),
            pltpu.VMEM((ch if deep else 8, width), jnp.float32),
            pltpu.VMEM((ch if deep else 8, width), jnp.float32),
            pltpu.VMEM_SHARED((npad, width), jnp.float32),
            pltpu.SemaphoreType.DMA,
            pltpu.SemaphoreType.DMA,
            pltpu.SemaphoreType.DMA,
            pltpu.SemaphoreType.DMA,
            pltpu.SemaphoreType.DMA,
            pltpu.SemaphoreType.DMA,
            pltpu.SemaphoreType.DMA,
            pltpu.SemaphoreType.DMA,
            pltpu.SemaphoreType.DMA,
        ],
        compiler_params=pltpu.CompilerParams(use_tc_tiling_on_sc=False),
    )


_BR = 1000


def _full(shape):
    return pl.BlockSpec(shape, lambda i: (0,) * len(shape))


def _rows(shape3=None, width=128):
    if shape3:
        return pl.BlockSpec((shape3[0], _BR, shape3[2]), lambda i: (0, i, 0))
    return pl.BlockSpec((_BR, width), lambda i: (i, 0))


def _ka_body(x_ref, w_ref, dc_ref, hp_ref, dis_ref):
    deg = 1.0 + dc_ref[0, :, 0:1] + dc_ref[1, :, 0:1]
    dis = lax.rsqrt(deg)
    h = jnp.dot(x_ref[...], w_ref[...], preferred_element_type=jnp.float32)
    hp_ref[...] = h * dis
    dis_ref[...] = dis


def _kb_body(act, pad_out, agg_ref, hp_ref, dis_ref, b_ref, w_ref, out_ref):
    dis = dis_ref[...]
    t = dis * (agg_ref[0] + agg_ref[1] + hp_ref[...]) + b_ref[...]
    t = act(t)
    v = dis * jnp.dot(t, w_ref[...], preferred_element_type=jnp.float32)
    if pad_out:
        v = jnp.concatenate([v, jnp.zeros_like(v)], axis=1)
    out_ref[...] = v


def _kc_body(agg_ref, hp_ref, dis_ref, b_ref, wc_ref, bc_ref, out_ref):
    t = (agg_ref[0] + agg_ref[1] + hp_ref[...])[:, :8]
    t = dis_ref[...] * t + b_ref[...]
    t = jnp.where(t >= 0, t, 0.01 * t)
    o = jnp.dot(t, wc_ref[...], preferred_element_type=jnp.float32) + bc_ref[...]
    out_ref[...] = jnp.where(o > 0, o, jnp.exp(o) - 1.0)


def _act_id(t):
    return t


def _act_tanh(t):
    return jnp.tanh(t)


def _act_leaky(t):
    return jnp.where(t >= 0, t, 0.01 * t)


def kernel(x, edge_index, W1, b1, W2, b2, W3, b3, W4, b4, Wc, bc):
    n, d = x.shape
    e = edge_index.shape[1]
    grid = (n // _BR,)
    npad = -(-n // (8 * _NS)) * (8 * _NS)
    nw = _NC * _NS
    ei = edge_index.astype(jnp.int32)
    ei_w = ei.reshape(2, nw, -1, _CHW)
    ei_n = ei.reshape(2, nw, -1, _CHN)
    ones_blk = jnp.ones((_CHN, 16), jnp.float32)
    zeros16 = jnp.zeros((npad, 16), jnp.float32)
    zerosd = jnp.zeros((npad, d), jnp.float32)

    deg16 = _make_agg(n, e, 16, False, _CHN)
    agg16 = _make_agg(n, e, 16, True, _CHN)
    aggd = _make_agg(n, e, d, True, _CHW)

    dc = deg16(ei_n, ones_blk, zeros16)

    hp1, dis = pl.pallas_call(
        _ka_body,
        grid=grid,
        in_specs=[_rows(width=d), _full((d, d)), _rows((_NC, n, 16))],
        out_specs=[_rows(width=d), _rows(width=1)],
        out_shape=[jax.ShapeDtypeStruct((n, d), jnp.float32),
                   jax.ShapeDtypeStruct((n, 1), jnp.float32)],
    )(x, W1, dc)

    def mid(aggp, hp, b, w, act, dout, pad_out):
        wout = 2 * dout if pad_out else dout
        return pl.pallas_call(
            functools.partial(_kb_body, act, pad_out),
            grid=grid,
            in_specs=[_rows((_NC, n, d)), _rows(width=d), _rows(width=1),
                      _full((1, d)), _full((d, dout))],
            out_specs=_rows(width=wout),
            out_shape=jax.ShapeDtypeStruct((n, wout), jnp.float32),
        )(aggp, hp, dis, b.reshape(1, d), w)

    a1 = aggd(ei_w, hp1, zerosd)
    hp2 = mid(a1, hp1, b1, W2, _act_id, d, False)
    a2 = aggd(ei_w, hp2, zerosd)
    hp3 = mid(a2, hp2, b2, W3, _act_tanh, d, False)
    a3 = aggd(ei_w, hp3, zerosd)
    hp4 = mid(a3, hp3, b3, W4, _act_leaky, 8, True)
    a4 = agg16(ei_n, hp4, zeros16)

    out = pl.pallas_call(
        _kc_body,
        grid=grid,
        in_specs=[_rows((_NC, n, 16)), _rows(width=16), _rows(width=1),
                  _full((1, 8)), _full((8, 1)), _full((1, 1))],
        out_specs=_rows(width=1),
        out_shape=jax.ShapeDtypeStruct((n, 1), jnp.float32),
    )(a4, hp4, dis, b4.reshape(1, 8), Wc, bc.reshape(1, 1))
    return out


# speedup vs baseline: 1.1342x; 1.1342x over previous
import functools

import jax
import jax.numpy as jnp
from jax import lax
from jax.experimental import pallas as pl
from jax.experimental.pallas import tpu as pltpu
from jax.experimental.pallas import tpu_sc as plsc

_NC = 2
_NS = 16
_CHW = 100
_CHN = 400


@functools.lru_cache(maxsize=None)
def _make_agg(n_nodes: int, n_edges: int, width: int, gather: bool, ch: int):
    nw = _NC * _NS
    epw = n_edges // nw
    assert epw * nw == n_edges and epw % ch == 0
    nit = epw // ch
    cpt = nit
    assert nit >= 3
    npad = -(-n_nodes // (8 * _NS)) * (8 * _NS)
    rpt = npad // _NS

    mesh = plsc.VectorSubcoreMesh(
        core_axis_name="c", subcore_axis_name="s",
        num_cores=_NC, num_subcores=_NS)

    deep = False

    def body(ei_hbm, tab_hbm, zeros_hbm, out_hbm,
             src_v, dst_v, rows_a, rows_b, rows_c, rows_d, acc_sh,
             gs0, gs1, gs2, gs3, ss0, ss1, ss2, ss3, sem_i):
        sem_a, sem_b = gs0, gs1
        c = lax.axis_index("c")
        s = lax.axis_index("s")
        wid = s * _NC + c
        idx = pltpu.async_copy(ei_hbm.at[1, wid], dst_v, sem_i)
        if gather:
            idx2 = pltpu.async_copy(ei_hbm.at[0, wid], src_v, sem_i)
        else:
            idx2 = None
            pltpu.sync_copy(tab_hbm, rows_a)
        pltpu.sync_copy(zeros_hbm.at[pl.ds(s * rpt, rpt)],
                        acc_sh.at[pl.ds(s * rpt, rpt)])
        idx.wait()
        if idx2 is not None:
            idx2.wait()
        plsc.subcore_barrier()

        if deep:
            bufs = [rows_a, rows_b, rows_c, rows_d]
            gsems = [gs0, gs1, gs2, gs3]
            ssems = [ss0, ss1, ss2, ss3]

            def gi(i, q):
                pltpu.async_copy(tab_hbm.at[src_v.at[i]], bufs[q], gsems[q])

            def gw(i, q):
                pltpu.make_async_copy(
                    tab_hbm.at[src_v.at[i]], bufs[q], gsems[q]).wait()

            def si(i, q):
                pltpu.async_copy(
                    bufs[q], acc_sh.at[dst_v.at[i]], ssems[q], add=True)

            def sw(i, q):
                pltpu.make_async_copy(
                    bufs[q], acc_sh.at[dst_v.at[i]], ssems[q]).wait()

            gi(0, 0)
            gi(1, 1)
            gw(0, 0); si(0, 0); gi(2, 2)
            gw(1, 1); si(1, 1); gi(3, 3)

            def quad(j, carry):
                i = 4 * j + 2
                for m in range(4):
                    q = (2 + m) % 4
                    gw(i + m, q)
                    si(i + m, q)
                    sw(i + m - 2, (q + 2) % 4)
                    gi(i + m + 2, (q + 2) % 4)
                return carry

            lax.fori_loop(0, (nit - 4) // 4, quad, 0)
            gw(nit - 2, 2); si(nit - 2, 2); sw(nit - 4, 0)
            gw(nit - 1, 3); si(nit - 1, 3); sw(nit - 3, 1)
            sw(nit - 2, 2)
            sw(nit - 1, 3)
        elif gather:
            def g_issue(i, buf, sem):
                pltpu.async_copy(tab_hbm.at[src_v.at[i]], buf, sem)

            def g_wait(i, buf, sem):
                pltpu.make_async_copy(tab_hbm.at[src_v.at[i]], buf, sem).wait()

            g_issue(0, rows_a, sem_a)

            def pair(j, carry):
                i = 2 * j
                g_issue(i + 1, rows_b, sem_b)
                g_wait(i, rows_a, sem_a)
                pltpu.sync_copy(rows_a, acc_sh.at[dst_v.at[i]], add=True)
                g_issue(i + 2, rows_a, sem_a)
                g_wait(i + 1, rows_b, sem_b)
                pltpu.sync_copy(rows_b, acc_sh.at[dst_v.at[i + 1]], add=True)
                return carry

            if nit % 2 == 1:
                lax.fori_loop(0, (nit - 1) // 2, pair, 0)
                g_wait(nit - 1, rows_a, sem_a)
                pltpu.sync_copy(rows_a, acc_sh.at[dst_v.at[nit - 1]], add=True)
            else:
                lax.fori_loop(0, nit // 2 - 1, pair, 0)
                g_issue(nit - 1, rows_b, sem_b)
                g_wait(nit - 2, rows_a, sem_a)
                pltpu.sync_copy(rows_a, acc_sh.at[dst_v.at[nit - 2]], add=True)
                g_wait(nit - 1, rows_b, sem_b)
                pltpu.sync_copy(rows_b, acc_sh.at[dst_v.at[nit - 1]], add=True)
        else:
            def step(i, carry):
                pltpu.sync_copy(rows_a, acc_sh.at[dst_v.at[i]], add=True)
                return carry

            lax.fori_loop(0, nit, step, 0)

        plsc.subcore_barrier()
        pltpu.sync_copy(acc_sh.at[pl.ds(s * rpt, rpt)],
                        out_hbm.at[c, pl.ds(s * rpt, rpt)])

    return pl.kernel(
        body,
        out_type=jax.ShapeDtypeStruct((_NC, npad, width), jnp.float32),
        mesh=mesh,
        scratch_types=[
            pltpu.VMEM((cpt, ch), jnp.int32),
            pltpu.VMEM((cpt, ch), jnp.int32),
            pltpu.VMEM((ch, width), jnp.float32),
            pltpu.VMEM((ch, width), jnp.float32),
            pltpu.VMEM((ch if deep else 8, width), jnp.float32),
            pltpu.VMEM((ch if deep else 8, width), jnp.float32),
            pltpu.VMEM_SHARED((npad, width), jnp.float32),
            pltpu.SemaphoreType.DMA,
            pltpu.SemaphoreType.DMA,
            pltpu.SemaphoreType.DMA,
            pltpu.SemaphoreType.DMA,
            pltpu.SemaphoreType.DMA,
            pltpu.SemaphoreType.DMA,
            pltpu.SemaphoreType.DMA,
            pltpu.SemaphoreType.DMA,
            pltpu.SemaphoreType.DMA,
        ],
        compiler_params=pltpu.CompilerParams(use_tc_tiling_on_sc=False),
    )


_BR = 1000


def _full(shape):
    return pl.BlockSpec(shape, lambda i: (0,) * len(shape))


def _rows(shape3=None, width=128):
    if shape3:
        return pl.BlockSpec((shape3[0], _BR, shape3[2]), lambda i: (0, i, 0))
    return pl.BlockSpec((_BR, width), lambda i: (i, 0))


def _ka_body(x_ref, w_ref, dc_ref, hp_ref, dis_ref):
    deg = 1.0 + dc_ref[0, :, 0:1] + dc_ref[1, :, 0:1]
    dis = lax.rsqrt(deg)
    h = jnp.dot(x_ref[...], w_ref[...], preferred_element_type=jnp.float32)
    hp_ref[...] = h * dis
    dis_ref[...] = dis


def _kb_body(act, pad_out, agg_ref, hp_ref, dis_ref, b_ref, w_ref, out_ref):
    dis = dis_ref[...]
    t = dis * (agg_ref[0] + agg_ref[1] + hp_ref[...]) + b_ref[...]
    t = act(t)
    v = dis * jnp.dot(t, w_ref[...], preferred_element_type=jnp.float32)
    if pad_out:
        v = jnp.concatenate([v, jnp.zeros_like(v)], axis=1)
    out_ref[...] = v


def _kc_body(agg_ref, hp_ref, dis_ref, b_ref, wc_ref, bc_ref, out_ref):
    t = (agg_ref[0] + agg_ref[1] + hp_ref[...])[:, :8]
    t = dis_ref[...] * t + b_ref[...]
    t = jnp.where(t >= 0, t, 0.01 * t)
    o = jnp.dot(t, wc_ref[...], preferred_element_type=jnp.float32) + bc_ref[...]
    out_ref[...] = jnp.where(o > 0, o, jnp.exp(o) - 1.0)


def _act_id(t):
    return t


def _act_tanh(t):
    return jnp.tanh(t)


def _act_leaky(t):
    return jnp.where(t >= 0, t, 0.01 * t)


def kernel(x, edge_index, W1, b1, W2, b2, W3, b3, W4, b4, Wc, bc):
    n, d = x.shape
    e = edge_index.shape[1]
    grid = (n // _BR,)
    npad = -(-n // (8 * _NS)) * (8 * _NS)
    nw = _NC * _NS
    ei = edge_index.astype(jnp.int32)
    ei_w = ei.reshape(2, nw, -1, _CHW)
    ei_n = ei.reshape(2, nw, -1, _CHN)
    ones_blk = jnp.ones((_CHN, 16), jnp.float32)
    zeros16 = jnp.zeros((npad, 16), jnp.float32)
    zerosd = jnp.zeros((npad, d), jnp.float32)

    deg16 = _make_agg(n, e, 16, False, _CHN)
    agg16 = _make_agg(n, e, 16, True, _CHN)
    aggd = _make_agg(n, e, d, True, _CHW)

    dc = deg16(ei_n, ones_blk, zeros16)

    hp1, dis = pl.pallas_call(
        _ka_body,
        grid=grid,
        in_specs=[_rows(width=d), _full((d, d)), _rows((_NC, n, 16))],
        out_specs=[_rows(width=d), _rows(width=1)],
        out_shape=[jax.ShapeDtypeStruct((n, d), jnp.float32),
                   jax.ShapeDtypeStruct((n, 1), jnp.float32)],
    )(x, W1, dc)

    def mid(aggp, hp, b, w, act, dout, pad_out):
        wout = 2 * dout if pad_out else dout
        return pl.pallas_call(
            functools.partial(_kb_body, act, pad_out),
            grid=grid,
            in_specs=[_rows((_NC, n, d)), _rows(width=d), _rows(width=1),
                      _full((1, d)), _full((d, dout))],
            out_specs=_rows(width=wout),
            out_shape=jax.ShapeDtypeStruct((n, wout), jnp.float32),
        )(aggp, hp, dis, b.reshape(1, d), w)

    a1 = aggd(ei_w, hp1, zerosd)
    hp2 = mid(a1, hp1, b1, W2, _act_id, d, False)
    a2 = aggd(ei_w, hp2, zerosd)
    hp3 = mid(a2, hp2, b2, W3, _act_tanh, d, False)
    a3 = aggd(ei_w, hp3, zerosd)
    hp4 = mid(a3, hp3, b3, W4, _act_leaky, 8, True)
    a4 = agg16(ei_n, hp4, zeros16)

    out = pl.pallas_call(
        _kc_body,
        grid=grid,
        in_specs=[_rows((_NC, n, 16)), _rows(width=16), _rows(width=1),
                  _full((1, 8)), _full((8, 1)), _full((1, 1))],
        out_specs=_rows(width=1),
        out_shape=jax.ShapeDtypeStruct((n, 1), jnp.float32),
    )(a4, hp4, dis, b4.reshape(1, 8), Wc, bc.reshape(1, 1))
    return out


# speedup vs baseline: 1.1582x; 1.0212x over previous
import functools

import jax
import jax.numpy as jnp
from jax import lax
from jax.experimental import pallas as pl
from jax.experimental.pallas import tpu as pltpu
from jax.experimental.pallas import tpu_sc as plsc

_NC = 2
_NS = 16
_CHW = 100
_CHN = 400


@functools.lru_cache(maxsize=None)
def _make_agg(n_nodes: int, n_edges: int, width: int, gather: bool, ch: int):
    nw = _NC * _NS
    epw = n_edges // nw
    assert epw * nw == n_edges and epw % ch == 0
    nit = epw // ch
    cpt = nit
    assert nit >= 3
    npad = -(-n_nodes // (8 * _NS)) * (8 * _NS)
    rpt = npad // _NS

    mesh = plsc.VectorSubcoreMesh(
        core_axis_name="c", subcore_axis_name="s",
        num_cores=_NC, num_subcores=_NS)

    deep = False

    def body(ei_hbm, tab_hbm, zeros_hbm, out_hbm,
             src_v, dst_v, rows_a, rows_b, rows_c, rows_d, acc_sh,
             gs0, gs1, gs2, gs3, ss0, ss1, ss2, ss3, sem_i):
        sem_a, sem_b = gs0, gs1
        c = lax.axis_index("c")
        s = lax.axis_index("s")
        wid = s * _NC + c
        idx = pltpu.async_copy(ei_hbm.at[1, wid], dst_v, sem_i)
        if gather:
            idx2 = pltpu.async_copy(ei_hbm.at[0, wid], src_v, sem_i)
        else:
            idx2 = None
            pltpu.sync_copy(tab_hbm, rows_a)
        pltpu.sync_copy(zeros_hbm.at[pl.ds(s * rpt, rpt)],
                        acc_sh.at[pl.ds(s * rpt, rpt)])
        idx.wait()
        if idx2 is not None:
            idx2.wait()
        plsc.subcore_barrier()

        if deep:
            bufs = [rows_a, rows_b, rows_c, rows_d]
            gsems = [gs0, gs1, gs2, gs3]
            ssems = [ss0, ss1, ss2, ss3]

            def gi(i, q):
                pltpu.async_copy(tab_hbm.at[src_v.at[i]], bufs[q], gsems[q])

            def gw(i, q):
                pltpu.make_async_copy(
                    tab_hbm.at[src_v.at[i]], bufs[q], gsems[q]).wait()

            def si(i, q):
                pltpu.async_copy(
                    bufs[q], acc_sh.at[dst_v.at[i]], ssems[q], add=True)

            def sw(i, q):
                pltpu.make_async_copy(
                    bufs[q], acc_sh.at[dst_v.at[i]], ssems[q]).wait()

            gi(0, 0)
            gi(1, 1)
            gw(0, 0); si(0, 0); gi(2, 2)
            gw(1, 1); si(1, 1); gi(3, 3)

            def quad(j, carry):
                i = 4 * j + 2
                for m in range(4):
                    q = (2 + m) % 4
                    gw(i + m, q)
                    si(i + m, q)
                    sw(i + m - 2, (q + 2) % 4)
                    gi(i + m + 2, (q + 2) % 4)
                return carry

            lax.fori_loop(0, (nit - 4) // 4, quad, 0)
            gw(nit - 2, 2); si(nit - 2, 2); sw(nit - 4, 0)
            gw(nit - 1, 3); si(nit - 1, 3); sw(nit - 3, 1)
            sw(nit - 2, 2)
            sw(nit - 1, 3)
        elif gather:
            def g_issue(i, buf, sem):
                pltpu.async_copy(tab_hbm.at[src_v.at[i]], buf, sem)

            def g_wait(i, buf, sem):
                pltpu.make_async_copy(tab_hbm.at[src_v.at[i]], buf, sem).wait()

            g_issue(0, rows_a, sem_a)

            def pair(j, carry):
                i = 2 * j
                g_issue(i + 1, rows_b, sem_b)
                g_wait(i, rows_a, sem_a)
                pltpu.sync_copy(rows_a, acc_sh.at[dst_v.at[i]], add=True)
                g_issue(i + 2, rows_a, sem_a)
                g_wait(i + 1, rows_b, sem_b)
                pltpu.sync_copy(rows_b, acc_sh.at[dst_v.at[i + 1]], add=True)
                return carry

            if nit % 2 == 1:
                lax.fori_loop(0, (nit - 1) // 2, pair, 0)
                g_wait(nit - 1, rows_a, sem_a)
                pltpu.sync_copy(rows_a, acc_sh.at[dst_v.at[nit - 1]], add=True)
            else:
                lax.fori_loop(0, nit // 2 - 1, pair, 0)
                g_issue(nit - 1, rows_b, sem_b)
                g_wait(nit - 2, rows_a, sem_a)
                pltpu.sync_copy(rows_a, acc_sh.at[dst_v.at[nit - 2]], add=True)
                g_wait(nit - 1, rows_b, sem_b)
                pltpu.sync_copy(rows_b, acc_sh.at[dst_v.at[nit - 1]], add=True)
        else:
            def step(i, carry):
                pltpu.sync_copy(rows_a, acc_sh.at[dst_v.at[i]], add=True)
                return carry

            lax.fori_loop(0, nit, step, 0)

        plsc.subcore_barrier()
        pltpu.sync_copy(acc_sh.at[pl.ds(s * rpt, rpt)],
                        out_hbm.at[c, pl.ds(s * rpt, rpt)])

    return pl.kernel(
        body,
        out_type=jax.ShapeDtypeStruct((_NC, npad, width), jnp.float32),
        mesh=mesh,
        scratch_types=[
            pltpu.VMEM((cpt, ch), jnp.int32),
            pltpu.VMEM((cpt, ch), jnp.int32),
            pltpu.VMEM((ch, width), jnp.float32),
            pltpu.VMEM((ch, width), jnp.float32),
            pltpu.VMEM((ch if deep else 8, width), jnp.float32),
            pltpu.VMEM((ch if deep else 8, width), jnp.float32),
            pltpu.VMEM_SHARED((npad, width), jnp.float32),
            pltpu.SemaphoreType.DMA,
            pltpu.SemaphoreType.DMA,
            pltpu.SemaphoreType.DMA,
            pltpu.SemaphoreType.DMA,
            pltpu.SemaphoreType.DMA,
            pltpu.SemaphoreType.DMA,
            pltpu.SemaphoreType.DMA,
            pltpu.SemaphoreType.DMA,
            pltpu.SemaphoreType.DMA,
        ],
        compiler_params=pltpu.CompilerParams(use_tc_tiling_on_sc=False),
    )


_BR = 2000


def _full(shape):
    return pl.BlockSpec(shape, lambda i: (0,) * len(shape))


def _rows(shape3=None, width=128):
    if shape3:
        return pl.BlockSpec((shape3[0], _BR, shape3[2]), lambda i: (0, i, 0))
    return pl.BlockSpec((_BR, width), lambda i: (i, 0))


def _ka_body(x_ref, w_ref, dc_ref, hp_ref, dis_ref):
    deg = 1.0 + dc_ref[0, :, 0:1] + dc_ref[1, :, 0:1]
    dis = lax.rsqrt(deg)
    h = jnp.dot(x_ref[...], w_ref[...], preferred_element_type=jnp.float32)
    hp_ref[...] = h * dis
    dis_ref[...] = dis


def _kb_body(act, pad_out, agg_ref, hp_ref, dis_ref, b_ref, w_ref, out_ref):
    dis = dis_ref[...]
    t = dis * (agg_ref[0] + agg_ref[1] + hp_ref[...]) + b_ref[...]
    t = act(t)
    v = dis * jnp.dot(t, w_ref[...], preferred_element_type=jnp.float32)
    if pad_out:
        v = jnp.concatenate([v, jnp.zeros_like(v)], axis=1)
    out_ref[...] = v


def _kc_body(agg_ref, hp_ref, dis_ref, b_ref, wc_ref, bc_ref, out_ref):
    t = (agg_ref[0] + agg_ref[1] + hp_ref[...])[:, :8]
    t = dis_ref[...] * t + b_ref[...]
    t = jnp.where(t >= 0, t, 0.01 * t)
    o = jnp.dot(t, wc_ref[...], preferred_element_type=jnp.float32) + bc_ref[...]
    out_ref[...] = jnp.where(o > 0, o, jnp.exp(o) - 1.0)


def _act_id(t):
    return t


def _act_tanh(t):
    return jnp.tanh(t)


def _act_leaky(t):
    return jnp.where(t >= 0, t, 0.01 * t)


def kernel(x, edge_index, W1, b1, W2, b2, W3, b3, W4, b4, Wc, bc):
    n, d = x.shape
    e = edge_index.shape[1]
    grid = (n // _BR,)
    npad = -(-n // (8 * _NS)) * (8 * _NS)
    nw = _NC * _NS
    ei = edge_index.astype(jnp.int32)
    ei_w = ei.reshape(2, nw, -1, _CHW)
    ei_n = ei.reshape(2, nw, -1, _CHN)
    ones_blk = jnp.ones((_CHN, 16), jnp.float32)
    zeros16 = jnp.zeros((npad, 16), jnp.float32)
    zerosd = jnp.zeros((npad, d), jnp.float32)

    deg16 = _make_agg(n, e, 16, False, _CHN)
    agg16 = _make_agg(n, e, 16, True, _CHN)
    aggd = _make_agg(n, e, d, True, _CHW)

    dc = deg16(ei_n, ones_blk, zeros16)

    hp1, dis = pl.pallas_call(
        _ka_body,
        grid=grid,
        in_specs=[_rows(width=d), _full((d, d)), _rows((_NC, n, 16))],
        out_specs=[_rows(width=d), _rows(width=1)],
        out_shape=[jax.ShapeDtypeStruct((n, d), jnp.float32),
                   jax.ShapeDtypeStruct((n, 1), jnp.float32)],
    )(x, W1, dc)

    def mid(aggp, hp, b, w, act, dout, pad_out):
        wout = 2 * dout if pad_out else dout
        return pl.pallas_call(
            functools.partial(_kb_body, act, pad_out),
            grid=grid,
            in_specs=[_rows((_NC, n, d)), _rows(width=d), _rows(width=1),
                      _full((1, d)), _full((d, dout))],
            out_specs=_rows(width=wout),
            out_shape=jax.ShapeDtypeStruct((n, wout), jnp.float32),
        )(aggp, hp, dis, b.reshape(1, d), w)

    a1 = aggd(ei_w, hp1, zerosd)
    hp2 = mid(a1, hp1, b1, W2, _act_id, d, False)
    a2 = aggd(ei_w, hp2, zerosd)
    hp3 = mid(a2, hp2, b2, W3, _act_tanh, d, False)
    a3 = aggd(ei_w, hp3, zerosd)
    hp4 = mid(a3, hp3, b3, W4, _act_leaky, 8, True)
    a4 = agg16(ei_n, hp4, zeros16)

    out = pl.pallas_call(
        _kc_body,
        grid=grid,
        in_specs=[_rows((_NC, n, 16)), _rows(width=16), _rows(width=1),
                  _full((1, 8)), _full((8, 1)), _full((1, 1))],
        out_specs=_rows(width=1),
        out_shape=jax.ShapeDtypeStruct((n, 1), jnp.float32),
    )(a4, hp4, dis, b4.reshape(1, 8), Wc, bc.reshape(1, 1))
    return out


# speedup vs baseline: 1.1676x; 1.0082x over previous
import functools

import jax
import jax.numpy as jnp
from jax import lax
from jax.experimental import pallas as pl
from jax.experimental.pallas import tpu as pltpu
from jax.experimental.pallas import tpu_sc as plsc

_NC = 2
_NS = 16
_CHW = 100
_CHN = 400


@functools.lru_cache(maxsize=None)
def _make_agg(n_nodes: int, n_edges: int, width: int, gather: bool, ch: int):
    nw = _NC * _NS
    epw = n_edges // nw
    assert epw * nw == n_edges and epw % ch == 0
    nit = epw // ch
    cpt = nit
    assert nit >= 3
    npad = -(-n_nodes // (8 * _NS)) * (8 * _NS)
    rpt = npad // _NS

    mesh = plsc.VectorSubcoreMesh(
        core_axis_name="c", subcore_axis_name="s",
        num_cores=_NC, num_subcores=_NS)

    deep = False

    def body(ei_hbm, tab_hbm, zeros_hbm, out_hbm,
             src_v, dst_v, rows_a, rows_b, rows_c, rows_d, acc_sh,
             gs0, gs1, gs2, gs3, ss0, ss1, ss2, ss3, sem_i):
        sem_a, sem_b = gs0, gs1
        c = lax.axis_index("c")
        s = lax.axis_index("s")
        wid = s * _NC + c
        idx = pltpu.async_copy(ei_hbm.at[1, wid], dst_v, sem_i)
        if gather:
            idx2 = pltpu.async_copy(ei_hbm.at[0, wid], src_v, sem_i)
        else:
            idx2 = None
            pltpu.sync_copy(tab_hbm, rows_a)
        pltpu.sync_copy(zeros_hbm.at[pl.ds(s * rpt, rpt)],
                        acc_sh.at[pl.ds(s * rpt, rpt)])
        idx.wait()
        if idx2 is not None:
            idx2.wait()
        plsc.subcore_barrier()

        if deep:
            bufs = [rows_a, rows_b, rows_c, rows_d]
            gsems = [gs0, gs1, gs2, gs3]
            ssems = [ss0, ss1, ss2, ss3]

            def gi(i, q):
                pltpu.async_copy(tab_hbm.at[src_v.at[i]], bufs[q], gsems[q])

            def gw(i, q):
                pltpu.make_async_copy(
                    tab_hbm.at[src_v.at[i]], bufs[q], gsems[q]).wait()

            def si(i, q):
                pltpu.async_copy(
                    bufs[q], acc_sh.at[dst_v.at[i]], ssems[q], add=True)

            def sw(i, q):
                pltpu.make_async_copy(
                    bufs[q], acc_sh.at[dst_v.at[i]], ssems[q]).wait()

            gi(0, 0)
            gi(1, 1)
            gw(0, 0); si(0, 0); gi(2, 2)
            gw(1, 1); si(1, 1); gi(3, 3)

            def quad(j, carry):
                i = 4 * j + 2
                for m in range(4):
                    q = (2 + m) % 4
                    gw(i + m, q)
                    si(i + m, q)
                    sw(i + m - 2, (q + 2) % 4)
                    gi(i + m + 2, (q + 2) % 4)
                return carry

            lax.fori_loop(0, (nit - 4) // 4, quad, 0)
            gw(nit - 2, 2); si(nit - 2, 2); sw(nit - 4, 0)
            gw(nit - 1, 3); si(nit - 1, 3); sw(nit - 3, 1)
            sw(nit - 2, 2)
            sw(nit - 1, 3)
        elif gather:
            def g_issue(i, buf, sem):
                pltpu.async_copy(tab_hbm.at[src_v.at[i]], buf, sem)

            def g_wait(i, buf, sem):
                pltpu.make_async_copy(tab_hbm.at[src_v.at[i]], buf, sem).wait()

            g_issue(0, rows_a, sem_a)

            def pair(j, carry):
                i = 2 * j
                g_issue(i + 1, rows_b, sem_b)
                g_wait(i, rows_a, sem_a)
                pltpu.sync_copy(rows_a, acc_sh.at[dst_v.at[i]], add=True)
                g_issue(i + 2, rows_a, sem_a)
                g_wait(i + 1, rows_b, sem_b)
                pltpu.sync_copy(rows_b, acc_sh.at[dst_v.at[i + 1]], add=True)
                return carry

            if nit % 2 == 1:
                lax.fori_loop(0, (nit - 1) // 2, pair, 0)
                g_wait(nit - 1, rows_a, sem_a)
                pltpu.sync_copy(rows_a, acc_sh.at[dst_v.at[nit - 1]], add=True)
            else:
                lax.fori_loop(0, nit // 2 - 1, pair, 0)
                g_issue(nit - 1, rows_b, sem_b)
                g_wait(nit - 2, rows_a, sem_a)
                pltpu.sync_copy(rows_a, acc_sh.at[dst_v.at[nit - 2]], add=True)
                g_wait(nit - 1, rows_b, sem_b)
                pltpu.sync_copy(rows_b, acc_sh.at[dst_v.at[nit - 1]], add=True)
        else:
            def step(i, carry):
                pltpu.sync_copy(rows_a, acc_sh.at[dst_v.at[i]], add=True)
                return carry

            lax.fori_loop(0, nit, step, 0)

        plsc.subcore_barrier()
        pltpu.sync_copy(acc_sh.at[pl.ds(s * rpt, rpt)],
                        out_hbm.at[c, pl.ds(s * rpt, rpt)])

    return pl.kernel(
        body,
        out_type=jax.ShapeDtypeStruct((_NC, npad, width), jnp.float32),
        mesh=mesh,
        scratch_types=[
            pltpu.VMEM((cpt, ch), jnp.int32),
            pltpu.VMEM((cpt, ch), jnp.int32),
            pltpu.VMEM((ch, width), jnp.float32),
            pltpu.VMEM((ch, width), jnp.float32),
            pltpu.VMEM((ch if deep else 8, width), jnp.float32),
            pltpu.VMEM((ch if deep else 8, width), jnp.float32),
            pltpu.VMEM_SHARED((npad, width), jnp.float32),
            pltpu.SemaphoreType.DMA,
            pltpu.SemaphoreType.DMA,
            pltpu.SemaphoreType.DMA,
            pltpu.SemaphoreType.DMA,
            pltpu.SemaphoreType.DMA,
            pltpu.SemaphoreType.DMA,
            pltpu.SemaphoreType.DMA,
            pltpu.SemaphoreType.DMA,
            pltpu.SemaphoreType.DMA,
        ],
        compiler_params=pltpu.CompilerParams(use_tc_tiling_on_sc=False),
    )


_BR = 5000


def _full(shape):
    return pl.BlockSpec(shape, lambda i: (0,) * len(shape))


def _rows(shape3=None, width=128):
    if shape3:
        return pl.BlockSpec((shape3[0], _BR, shape3[2]), lambda i: (0, i, 0))
    return pl.BlockSpec((_BR, width), lambda i: (i, 0))


def _ka_body(x_ref, w_ref, dc_ref, hp_ref, dis_ref):
    deg = 1.0 + dc_ref[0, :, 0:1] + dc_ref[1, :, 0:1]
    dis = lax.rsqrt(deg)
    h = jnp.dot(x_ref[...], w_ref[...], preferred_element_type=jnp.float32)
    hp_ref[...] = h * dis
    dis_ref[...] = dis


def _kb_body(act, pad_out, agg_ref, hp_ref, dis_ref, b_ref, w_ref, out_ref):
    dis = dis_ref[...]
    t = dis * (agg_ref[0] + agg_ref[1] + hp_ref[...]) + b_ref[...]
    t = act(t)
    v = dis * jnp.dot(t, w_ref[...], preferred_element_type=jnp.float32)
    if pad_out:
        v = jnp.concatenate([v, jnp.zeros_like(v)], axis=1)
    out_ref[...] = v


def _kc_body(agg_ref, hp_ref, dis_ref, b_ref, wc_ref, bc_ref, out_ref):
    t = (agg_ref[0] + agg_ref[1] + hp_ref[...])[:, :8]
    t = dis_ref[...] * t + b_ref[...]
    t = jnp.where(t >= 0, t, 0.01 * t)
    o = jnp.dot(t, wc_ref[...], preferred_element_type=jnp.float32) + bc_ref[...]
    out_ref[...] = jnp.where(o > 0, o, jnp.exp(o) - 1.0)


def _act_id(t):
    return t


def _act_tanh(t):
    return jnp.tanh(t)


def _act_leaky(t):
    return jnp.where(t >= 0, t, 0.01 * t)


def kernel(x, edge_index, W1, b1, W2, b2, W3, b3, W4, b4, Wc, bc):
    n, d = x.shape
    e = edge_index.shape[1]
    grid = (n // _BR,)
    npad = -(-n // (8 * _NS)) * (8 * _NS)
    nw = _NC * _NS
    ei = edge_index.astype(jnp.int32)
    ei_w = ei.reshape(2, nw, -1, _CHW)
    ei_n = ei.reshape(2, nw, -1, _CHN)
    ones_blk = jnp.ones((_CHN, 16), jnp.float32)
    zeros16 = jnp.zeros((npad, 16), jnp.float32)
    zerosd = jnp.zeros((npad, d), jnp.float32)

    deg16 = _make_agg(n, e, 16, False, _CHN)
    agg16 = _make_agg(n, e, 16, True, _CHN)
    aggd = _make_agg(n, e, d, True, _CHW)

    dc = deg16(ei_n, ones_blk, zeros16)

    hp1, dis = pl.pallas_call(
        _ka_body,
        grid=grid,
        in_specs=[_rows(width=d), _full((d, d)), _rows((_NC, n, 16))],
        out_specs=[_rows(width=d), _rows(width=1)],
        out_shape=[jax.ShapeDtypeStruct((n, d), jnp.float32),
                   jax.ShapeDtypeStruct((n, 1), jnp.float32)],
    )(x, W1, dc)

    def mid(aggp, hp, b, w, act, dout, pad_out):
        wout = 2 * dout if pad_out else dout
        return pl.pallas_call(
            functools.partial(_kb_body, act, pad_out),
            grid=grid,
            in_specs=[_rows((_NC, n, d)), _rows(width=d), _rows(width=1),
                      _full((1, d)), _full((d, dout))],
            out_specs=_rows(width=wout),
            out_shape=jax.ShapeDtypeStruct((n, wout), jnp.float32),
        )(aggp, hp, dis, b.reshape(1, d), w)

    a1 = aggd(ei_w, hp1, zerosd)
    hp2 = mid(a1, hp1, b1, W2, _act_id, d, False)
    a2 = aggd(ei_w, hp2, zerosd)
    hp3 = mid(a2, hp2, b2, W3, _act_tanh, d, False)
    a3 = aggd(ei_w, hp3, zerosd)
    hp4 = mid(a3, hp3, b3, W4, _act_leaky, 8, True)
    a4 = agg16(ei_n, hp4, zeros16)

    out = pl.pallas_call(
        _kc_body,
        grid=grid,
        in_specs=[_rows((_NC, n, 16)), _rows(width=16), _rows(width=1),
                  _full((1, 8)), _full((8, 1)), _full((1, 1))],
        out_specs=_rows(width=1),
        out_shape=jax.ShapeDtypeStruct((n, 1), jnp.float32),
    )(a4, hp4, dis, b4.reshape(1, 8), Wc, bc.reshape(1, 1))
    return out
